# baseline (speedup 1.0000x reference)
import functools

import jax
import jax.numpy as jnp
from jax import lax
from jax.experimental import pallas as pl
from jax.experimental.pallas import tpu as pltpu
from jax.experimental.pallas import tpu_sc as plsc

N = 10000
E = 640000
K = 8
S = 1500
SPAD = 1536
NW = 32
EPW = E // NW
ECH = 2000
RB = 128
NRB = SPAD // RB


def _prep_body(w_ref, negg_ref, z_ref, gt_ref, mask_ref, vt_ref):
    w = w_ref[...]
    negg = negg_ref[...]
    p = w / jnp.sum(w)
    g = negg - jnp.log(p)
    bu = lax.bitcast_convert_type(g, jnp.uint32)
    key = jnp.where(bu >> 31 == 1, ~bu, bu | jnp.uint32(0x80000000))

    def bs_body(_, carry):
        lo, hi = carry
        mid = lo + (hi - lo) // 2
        cnt = jnp.sum((key <= mid).astype(jnp.int32))
        take = cnt >= S
        return jnp.where(take, lo, mid + 1), jnp.where(take, mid, hi)

    _, thr = lax.fori_loop(
        0, 32, bs_body, (jnp.uint32(0), jnp.uint32(0xFFFFFFFF)))
    mask = (key <= thr).astype(jnp.float32)

    z = z_ref[...]
    ze = jnp.exp(z - jnp.max(z, axis=0, keepdims=True))
    zs = ze / jnp.sum(ze, axis=0, keepdims=True)
    gt = gt_ref[...]
    gs = 1.0 / (1.0 + jnp.exp(-gt))
    zgt = zs * gs
    ct = zgt / jnp.sum(zgt, axis=1, keepdims=True)
    zsm = zs * mask
    m2 = lax.dot_general(zsm, ct, (((1,), (1,)), ((), ())),
                         preferred_element_type=jnp.float32)
    vt = lax.dot_general(m2, zs, (((1,), (0,)), ((), ())),
                         preferred_element_type=jnp.float32)
    mask_ref[...] = mask
    vt_ref[...] = vt


_prep_call = pl.pallas_call(
    _prep_body,
    out_shape=[
        jax.ShapeDtypeStruct((1, N), jnp.float32),
        jax.ShapeDtypeStruct((K, N), jnp.float32),
    ],
)


def _sc_sqrt(x):
    b = plsc.bitcast(x, jnp.int32)
    y = plsc.bitcast((b >> 1) + jnp.int32(0x1FBD1DF5), jnp.float32)
    y = 0.5 * (y + x / y)
    y = 0.5 * (y + x / y)
    return y


def _sc_body(vt_hbm, beta_hbm, mask_hbm, ii_hbm, jj_hbm,
             vsamp_hbm, bsamp_hbm, acc_hbm,
             vt_v, beta_v, mask_v, ich_v, jch_v, si_v, sj_v, sid_v, vs_v, bs_v,
             st_v):
    wid = lax.axis_index("s") * 2 + lax.axis_index("c")
    pltpu.sync_copy(vt_hbm, vt_v)
    pltpu.sync_copy(beta_hbm, beta_v)
    pltpu.sync_copy(mask_hbm, mask_v)

    @pl.when(wid == 0)
    def _():
        def z_body(i, c):
            sid_v[pl.ds(i * 16, 16)] = jnp.zeros((16,), jnp.int32)
            return c

        lax.fori_loop(0, SPAD // 16, z_body, 0)

        def comp_body(i, c):
            mv = mask_v[pl.ds(i * 16, 16)]
            sel = mv > 0.5
            seli = sel.astype(jnp.int32)
            pos = c + plsc.cumsum(seli) - 1
            ids = lax.iota(jnp.int32, 16) + i * 16
            okm = sel & (pos < SPAD)
            plsc.store_scatter(sid_v, [pos], ids, mask=okm)
            return c + jnp.sum(seli)

        cnt = lax.fori_loop(0, N // 16, comp_body, jnp.int32(0))

        def gath_body(j, c):
            off = j * 16
            idxv = sid_v[pl.ds(off, 16)]
            posv = lax.iota(jnp.int32, 16) + off
            valid = posv < cnt
            bk = plsc.load_gather(beta_v, [idxv])
            bs_v[pl.ds(off, 16)] = jnp.where(valid, bk, -1e9)
            for k in range(K):
                kk = jnp.full((16,), k, jnp.int32)
                vs_v[pl.ds(k * SPAD + off, 16)] = plsc.load_gather(vt_v, [kk, idxv])
            return c

        lax.fori_loop(0, SPAD // 16, gath_body, 0)
        pltpu.sync_copy(vs_v, vsamp_hbm)
        pltpu.sync_copy(bs_v, bsamp_hbm)

    ebase = wid * EPW

    def chunk_body(ci, carry):
        accb, accd = carry
        base = ebase + ci * ECH
        pltpu.sync_copy(ii_hbm.at[pl.ds(base, ECH)], ich_v)
        pltpu.sync_copy(jj_hbm.at[pl.ds(base, ECH)], jch_v)

        def pass1_body(vi, carry2):
            ab, ec = carry2
            off = vi * 16
            ii = ich_v[pl.ds(off, 16)]
            jj = jch_v[pl.ds(off, 16)]
            mi = plsc.load_gather(mask_v, [ii])
            mj = plsc.load_gather(mask_v, [jj])
            keep = mi * mj
            kb = keep > 0.5
            bsum = plsc.load_gather(beta_v, [ii]) + plsc.load_gather(beta_v, [jj])
            pos = ec + plsc.cumsum(kb.astype(jnp.int32)) - 1
            plsc.store_scatter(si_v, [pos], ii, mask=kb)
            plsc.store_scatter(sj_v, [pos], jj, mask=kb)
            return ab + keep * bsum, ec + jnp.sum(kb.astype(jnp.int32))

        accb, ec = lax.fori_loop(0, ECH // 16, pass1_body,
                                 (accb, jnp.int32(0)))

        def pass2_body(vi, ad):
            off = vi * 16
            lane = lax.iota(jnp.int32, 16) + off
            valid = lane < ec
            ii = jnp.where(valid, si_v[pl.ds(off, 16)], 0)
            jj = jnp.where(valid, sj_v[pl.ds(off, 16)], 0)
            d2 = jnp.zeros((16,), jnp.float32)
            for k in range(K):
                kk = jnp.full((16,), k, jnp.int32)
                d = (plsc.load_gather(vt_v, [kk, ii])
                     - plsc.load_gather(vt_v, [kk, jj]) + 1e-6)
                d2 = d2 + d * d
            return ad + jnp.where(valid, _sc_sqrt(d2), 0.0)

        nv = (ec + 15) // 16
        accd = lax.fori_loop(0, nv, pass2_body, accd)
        return accb, accd

    accb, accd = lax.fori_loop(
        0, EPW // ECH, chunk_body,
        (jnp.zeros((16,), jnp.float32), jnp.zeros((16,), jnp.float32)))
    st_v[pl.ds(0, 16)] = accb
    st_v[pl.ds(16, 16)] = accd
    pltpu.sync_copy(st_v, acc_hbm.at[pl.ds(wid * 32, 32)])


_sc_call = pl.kernel(
    _sc_body,
    out_type=[
        jax.ShapeDtypeStruct((K * SPAD,), jnp.float32),
        jax.ShapeDtypeStruct((SPAD,), jnp.float32),
        jax.ShapeDtypeStruct((NW * 32,), jnp.float32),
    ],
    mesh=plsc.VectorSubcoreMesh(core_axis_name="c", subcore_axis_name="s"),
    scratch_types=[
        pltpu.VMEM((K, N), jnp.float32),
        pltpu.VMEM((N,), jnp.float32),
        pltpu.VMEM((N,), jnp.float32),
        pltpu.VMEM((ECH,), jnp.int32),
        pltpu.VMEM((ECH,), jnp.int32),
        pltpu.VMEM((ECH,), jnp.int32),
        pltpu.VMEM((ECH,), jnp.int32),
        pltpu.VMEM((SPAD,), jnp.int32),
        pltpu.VMEM((K * SPAD,), jnp.float32),
        pltpu.VMEM((SPAD,), jnp.float32),
        pltpu.VMEM((32,), jnp.float32),
    ],
    compiler_params=pltpu.CompilerParams(needs_layout_passes=False),
)


def _dense_body(vs_ref, bs_ref, acc_ref, a_ref, out_ref):
    i = pl.program_id(0)
    at_full = vs_ref[...]
    bs = bs_ref[...]
    atr = vs_ref[:, pl.ds(i * RB, RB)]
    bsr = bs_ref[:, pl.ds(i * RB, RB)]

    ones_t = jnp.ones((1, SPAD), jnp.float32)
    cdims = (((0,), (0,)), ((), ()))
    nt = jnp.sum(at_full * at_full, axis=0, keepdims=True)
    rt = jnp.sum(at_full, axis=0, keepdims=True)
    nr = lax.dot_general(jnp.sum(atr * atr, axis=0, keepdims=True), ones_t,
                         cdims, preferred_element_type=jnp.float32)
    rr = lax.dot_general(jnp.sum(atr, axis=0, keepdims=True), ones_t,
                         cdims, preferred_element_type=jnp.float32)
    br = lax.dot_general(bsr, ones_t, cdims,
                         preferred_element_type=jnp.float32)
    p = lax.dot_general(atr, at_full, cdims,
                        preferred_element_type=jnp.float32)

    a = a_ref[0]
    sa = jnp.maximum(a, 0.0) + jnp.log(1.0 + jnp.exp(-jnp.abs(a)))

    d2 = nr + nt - 2.0 * p + 2e-6 * (rr - rt) + 8e-12
    dist = jnp.sqrt(jnp.maximum(d2, 0.0))
    mat = jnp.exp(br + bs - sa * dist)
    rowi = lax.broadcasted_iota(jnp.int32, (RB, SPAD), 0) + i * RB
    coli = lax.broadcasted_iota(jnp.int32, (RB, SPAD), 1)
    s_off = jnp.sum(jnp.where(rowi == coli, 0.0, mat))
    e1 = jnp.exp(jnp.float32(1.0))
    part = 0.5 * (e1 * e1) * s_off

    @pl.when(i == 0)
    def _():
        sb = jnp.sum(acc_ref[:, 0:16])
        sd = jnp.sum(acc_ref[:, 16:32])
        out_ref[0, 0] = (sb - sa * sd) - part

    @pl.when(i > 0)
    def _():
        out_ref[0, 0] = out_ref[0, 0] - part


_dense_call = pl.pallas_call(
    _dense_body,
    grid=(NRB,),
    in_specs=[
        pl.BlockSpec((K, SPAD), lambda i: (0, 0)),
        pl.BlockSpec((1, SPAD), lambda i: (0, 0)),
        pl.BlockSpec((NW, 32), lambda i: (0, 0)),
        pl.BlockSpec(memory_space=pltpu.SMEM),
    ],
    out_specs=pl.BlockSpec(memory_space=pltpu.SMEM),
    out_shape=jax.ShapeDtypeStruct((1, 1), jnp.float32),
)


def kernel(sampling_weights, sparse_i_idx, sparse_j_idx, beta, a, Z, G):
    negg = -jax.random.gumbel(jax.random.key(123), (N,), jnp.float32)
    maskf, vt = _prep_call(
        sampling_weights.reshape(1, N), negg.reshape(1, N), Z, G.T)
    vsamp, bsamp, acc = _sc_call(
        vt, beta, maskf.reshape(N), sparse_i_idx, sparse_j_idx)
    out = _dense_call(vsamp.reshape(K, SPAD), bsamp.reshape(1, SPAD),
                      acc.reshape(NW, 32), a)
    return out[0, 0]


# speedup vs baseline: 1.2865x; 1.2865x over previous
import functools

import jax
import jax.numpy as jnp
from jax import lax
from jax.experimental import pallas as pl
from jax.experimental.pallas import tpu as pltpu
from jax.experimental.pallas import tpu_sc as plsc

N = 10000
E = 640000
K = 8
S = 1500
SPAD = 1536
NW = 32
ECH = 2560
NCH = 8
E0 = 2 * ECH
U = 2
NIT = ECH // (16 * U)
RB = 128
NRB = SPAD // RB


def _prep_body(w_ref, negg_ref, z_ref, gt_ref, mask_ref, vt_ref):
    w = w_ref[...]
    negg = negg_ref[...]
    p = w / jnp.sum(w)
    g = negg - jnp.log(p)
    bu = lax.bitcast_convert_type(g, jnp.uint32)
    key = jnp.where(bu >> 31 == 1, ~bu, bu | jnp.uint32(0x80000000))

    def bs_body(_, carry):
        lo, hi = carry
        mid = lo + (hi - lo) // 2
        cnt = jnp.sum((key <= mid).astype(jnp.int32))
        take = cnt >= S
        return jnp.where(take, lo, mid + 1), jnp.where(take, mid, hi)

    _, thr = lax.fori_loop(
        0, 32, bs_body, (jnp.uint32(0), jnp.uint32(0xFFFFFFFF)))
    mask = (key <= thr).astype(jnp.float32)

    z = z_ref[...]
    ze = jnp.exp(z - jnp.max(z, axis=0, keepdims=True))
    zs = ze / jnp.sum(ze, axis=0, keepdims=True)
    gt = gt_ref[...]
    gs = 1.0 / (1.0 + jnp.exp(-gt))
    zgt = zs * gs
    ct = zgt / jnp.sum(zgt, axis=1, keepdims=True)
    zsm = zs * mask
    m2 = lax.dot_general(zsm, ct, (((1,), (1,)), ((), ())),
                         preferred_element_type=jnp.float32)
    vt = lax.dot_general(m2, zs, (((1,), (0,)), ((), ())),
                         preferred_element_type=jnp.float32)
    mask_ref[...] = mask
    vt_ref[...] = vt


_prep_call = pl.pallas_call(
    _prep_body,
    out_shape=[
        jax.ShapeDtypeStruct((1, N), jnp.float32),
        jax.ShapeDtypeStruct((K, N), jnp.float32),
    ],
)


def _sc_sqrt(x):
    b = plsc.bitcast(x, jnp.int32)
    y = plsc.bitcast((b >> 1) + jnp.int32(0x1FBD1DF5), jnp.float32)
    return 0.5 * (y + x / y)


def _sc_body(vt_hbm, beta_hbm, mask_hbm, ii_hbm, jj_hbm,
             vsamp_hbm, bsamp_hbm, acc_hbm,
             vt_v, beta_v, mask_v, ia_v, ja_v, ib_v, jb_v, sid_v, vs_v, bs_v,
             st_v, accb_v, accd_v, sia, sja, sib, sjb):
    wid = lax.axis_index("s") * 2 + lax.axis_index("c")
    pltpu.sync_copy(vt_hbm, vt_v)
    pltpu.sync_copy(beta_hbm, beta_v)
    pltpu.sync_copy(mask_hbm, mask_v)

    @pl.when(wid == 0)
    def _():
        def z_body(i, c):
            sid_v[pl.ds(i * 16, 16)] = jnp.zeros((16,), jnp.int32)
            return c

        lax.fori_loop(0, SPAD // 16, z_body, 0)

        def comp_body(i, c):
            mv = mask_v[pl.ds(i * 16, 16)]
            sel = mv > 0.5
            seli = sel.astype(jnp.int32)
            cum = plsc.cumsum(seli)
            pos = c + cum - 1
            ids = lax.iota(jnp.int32, 16) + i * 16
            okm = sel & (pos < SPAD)
            plsc.store_scatter(sid_v, [pos], ids, mask=okm)
            return c + cum[15]

        cnt = lax.fori_loop(0, N // 16, comp_body, jnp.int32(0))

        def gath_body(j, c):
            off = j * 16
            idxv = sid_v[pl.ds(off, 16)]
            posv = lax.iota(jnp.int32, 16) + off
            valid = posv < cnt
            bk = plsc.load_gather(beta_v, [idxv])
            bs_v[pl.ds(off, 16)] = jnp.where(valid, bk, -1e9)
            for k in range(K):
                kk = jnp.full((16,), k, jnp.int32)
                vs_v[pl.ds(k * SPAD + off, 16)] = plsc.load_gather(vt_v, [kk, idxv])
            return c

        lax.fori_loop(0, SPAD // 16, gath_body, 0)
        pltpu.sync_copy(vs_v, vsamp_hbm)
        pltpu.sync_copy(bs_v, bsamp_hbm)

    nch = jnp.where(wid == 0, 2, NCH)
    tbase = jnp.where(wid == 0, 0, E0 + (wid - 1) * (NCH * ECH))

    def _start(buf_i, buf_j, sem_i, sem_j, ci):
        base = tbase + ci * ECH
        pltpu.make_async_copy(ii_hbm.at[pl.ds(base, ECH)], buf_i, sem_i).start()
        pltpu.make_async_copy(jj_hbm.at[pl.ds(base, ECH)], buf_j, sem_j).start()

    def _wait(buf_i, buf_j, sem_i, sem_j):
        pltpu.make_async_copy(ii_hbm.at[pl.ds(0, ECH)], buf_i, sem_i).wait()
        pltpu.make_async_copy(jj_hbm.at[pl.ds(0, ECH)], buf_j, sem_j).wait()

    def _vreg(ich, jch, off, ab, ad):
        ii = ich[pl.ds(off, 16)]
        jj = jch[pl.ds(off, 16)]
        keep = plsc.load_gather(mask_v, [ii]) * plsc.load_gather(mask_v, [jj])
        bsum = plsc.load_gather(beta_v, [ii]) + plsc.load_gather(beta_v, [jj])
        d2 = jnp.zeros((16,), jnp.float32)
        for k in range(K):
            kk = jnp.full((16,), k, jnp.int32)
            d = (plsc.load_gather(vt_v, [kk, ii])
                 - plsc.load_gather(vt_v, [kk, jj]) + 1e-6)
            d2 = d2 + d * d
        return ab + keep * bsum, ad + keep * _sc_sqrt(d2)

    def _chunk(ich, jch):
        def it_body(it, carry2):
            ab, ad = carry2
            for u in range(U):
                ab, ad = _vreg(ich, jch, it * (16 * U) + u * 16, ab, ad)
            return ab, ad

        a0 = jnp.zeros((16,), jnp.float32)
        ab, ad = lax.fori_loop(0, NIT, it_body, (a0, a0))
        accb_v[...] = accb_v[...] + ab
        accd_v[...] = accd_v[...] + ad

    accb_v[...] = jnp.zeros((16,), jnp.float32)
    accd_v[...] = jnp.zeros((16,), jnp.float32)
    _start(ia_v, ja_v, sia, sja, 0)
    for ci in range(NCH):
        cur_i, cur_j = (ia_v, ja_v) if ci % 2 == 0 else (ib_v, jb_v)
        csi, csj = (sia, sja) if ci % 2 == 0 else (sib, sjb)
        nxt_i, nxt_j = (ib_v, jb_v) if ci % 2 == 0 else (ia_v, ja_v)
        nsi, nsj = (sib, sjb) if ci % 2 == 0 else (sia, sja)

        @pl.when(ci < nch)
        def _():
            _wait(cur_i, cur_j, csi, csj)

            @pl.when(ci + 1 < nch)
            def _():
                _start(nxt_i, nxt_j, nsi, nsj, ci + 1)

            _chunk(cur_i, cur_j)

    accb = accb_v[...]
    accd = accd_v[...]
    st_v[pl.ds(0, 16)] = accb
    st_v[pl.ds(16, 16)] = accd
    pltpu.sync_copy(st_v, acc_hbm.at[pl.ds(wid * 32, 32)])


_sc_call = pl.kernel(
    _sc_body,
    out_type=[
        jax.ShapeDtypeStruct((K * SPAD,), jnp.float32),
        jax.ShapeDtypeStruct((SPAD,), jnp.float32),
        jax.ShapeDtypeStruct((NW * 32,), jnp.float32),
    ],
    mesh=plsc.VectorSubcoreMesh(core_axis_name="c", subcore_axis_name="s"),
    scratch_types=[
        pltpu.VMEM((K, N), jnp.float32),
        pltpu.VMEM((N,), jnp.float32),
        pltpu.VMEM((N,), jnp.float32),
        pltpu.VMEM((ECH,), jnp.int32),
        pltpu.VMEM((ECH,), jnp.int32),
        pltpu.VMEM((ECH,), jnp.int32),
        pltpu.VMEM((ECH,), jnp.int32),
        pltpu.VMEM((SPAD,), jnp.int32),
        pltpu.VMEM((K * SPAD,), jnp.float32),
        pltpu.VMEM((SPAD,), jnp.float32),
        pltpu.VMEM((32,), jnp.float32),
        pltpu.VMEM((16,), jnp.float32),
        pltpu.VMEM((16,), jnp.float32),
        pltpu.SemaphoreType.DMA,
        pltpu.SemaphoreType.DMA,
        pltpu.SemaphoreType.DMA,
        pltpu.SemaphoreType.DMA,
    ],
    compiler_params=pltpu.CompilerParams(needs_layout_passes=False),
)


def _dense_body(vs_ref, bs_ref, acc_ref, a_ref, out_ref):
    i = pl.program_id(0)
    at_full = vs_ref[...]
    bs = bs_ref[...]
    atr = vs_ref[:, pl.ds(i * RB, RB)]
    bsr = bs_ref[:, pl.ds(i * RB, RB)]

    ones_t = jnp.ones((1, SPAD), jnp.float32)
    cdims = (((0,), (0,)), ((), ()))
    nt = jnp.sum(at_full * at_full, axis=0, keepdims=True)
    rt = jnp.sum(at_full, axis=0, keepdims=True)
    nr = lax.dot_general(jnp.sum(atr * atr, axis=0, keepdims=True), ones_t,
                         cdims, preferred_element_type=jnp.float32)
    rr = lax.dot_general(jnp.sum(atr, axis=0, keepdims=True), ones_t,
                         cdims, preferred_element_type=jnp.float32)
    br = lax.dot_general(bsr, ones_t, cdims,
                         preferred_element_type=jnp.float32)
    p = lax.dot_general(atr, at_full, cdims,
                        preferred_element_type=jnp.float32)

    a = a_ref[0]
    sa = jnp.maximum(a, 0.0) + jnp.log(1.0 + jnp.exp(-jnp.abs(a)))

    d2 = nr + nt - 2.0 * p + 2e-6 * (rr - rt) + 8e-12
    dist = jnp.sqrt(jnp.maximum(d2, 0.0))
    mat = jnp.exp(br + bs - sa * dist)
    rowi = lax.broadcasted_iota(jnp.int32, (RB, SPAD), 0) + i * RB
    coli = lax.broadcasted_iota(jnp.int32, (RB, SPAD), 1)
    s_off = jnp.sum(jnp.where(rowi == coli, 0.0, mat))
    e1 = jnp.exp(jnp.float32(1.0))
    part = 0.5 * (e1 * e1) * s_off

    @pl.when(i == 0)
    def _():
        sb = jnp.sum(acc_ref[:, 0:16])
        sd = jnp.sum(acc_ref[:, 16:32])
        out_ref[0, 0] = (sb - sa * sd) - part

    @pl.when(i > 0)
    def _():
        out_ref[0, 0] = out_ref[0, 0] - part


_dense_call = pl.pallas_call(
    _dense_body,
    grid=(NRB,),
    in_specs=[
        pl.BlockSpec((K, SPAD), lambda i: (0, 0)),
        pl.BlockSpec((1, SPAD), lambda i: (0, 0)),
        pl.BlockSpec((NW, 32), lambda i: (0, 0)),
        pl.BlockSpec(memory_space=pltpu.SMEM),
    ],
    out_specs=pl.BlockSpec(memory_space=pltpu.SMEM),
    out_shape=jax.ShapeDtypeStruct((1, 1), jnp.float32),
)


def kernel(sampling_weights, sparse_i_idx, sparse_j_idx, beta, a, Z, G):
    negg = -jax.random.gumbel(jax.random.key(123), (N,), jnp.float32)
    maskf, vt = _prep_call(
        sampling_weights.reshape(1, N), negg.reshape(1, N), Z, G.T)
    vsamp, bsamp, acc = _sc_call(
        vt, beta, maskf.reshape(N), sparse_i_idx, sparse_j_idx)
    out = _dense_call(vsamp.reshape(K, SPAD), bsamp.reshape(1, SPAD),
                      acc.reshape(NW, 32), a)
    return out[0, 0]
